# baseline (speedup 1.0000x reference)
import functools

import jax
import jax.numpy as jnp
from jax import lax
from jax.experimental import pallas as pl
from jax.experimental.pallas import tpu as pltpu
from jax.experimental.pallas import tpu_sc as plsc

_NC = 2
_NS = 16
_NW = _NC * _NS
_L = 16
_SG = 128


def _make_gather(batch, vocab, dim):
    b_per_w = batch // _NW
    n_sg = b_per_w // _SG
    mesh = plsc.VectorSubcoreMesh(core_axis_name="c", subcore_axis_name="s")

    @functools.partial(
        pl.kernel,
        mesh=mesh,
        out_type=jax.ShapeDtypeStruct((batch, dim), jnp.float32),
        scratch_types=[
            pltpu.VMEM((b_per_w,), jnp.int32),
            pltpu.VMEM((n_sg, _SG), jnp.int32),
            pltpu.VMEM((b_per_w // 2, 2 * dim), jnp.float32),
            pltpu.VMEM((b_per_w, dim), jnp.float32),
            pltpu.SemaphoreType.DMA,
        ],
    )
    def gather_kernel(idx_hbm, pairs_hbm, out_hbm, idx_v, pidx_v, pairs_v,
                      rows_v, sem):
        wid = lax.axis_index("s") * _NC + lax.axis_index("c")
        base = wid * b_per_w
        pltpu.sync_copy(idx_hbm.at[pl.ds(base, b_per_w)], idx_v)

        for c in range(n_sg):
            for b in range(_SG // _L):
                v = idx_v[pl.ds(c * _SG + b * _L, _L)]
                pidx_v[c, pl.ds(b * _L, _L)] = v >> 1

        def select_block(b, carry):
            par = idx_v[pl.ds(carry + b * _L, _L)] & 1
            for j in range(_L):
                row = b * _L + j
                off = par[j] * dim
                for k in range(dim // _L):
                    rows_v[carry + row, pl.ds(k * _L, _L)] = (
                        pairs_v[row, pl.ds(off + k * _L, _L)]
                    )
            return carry

        half_sg = n_sg // 2
        for h in range(2):
            copies = [
                pltpu.async_copy(
                    pairs_hbm.at[pidx_v.at[h * half_sg + c]],
                    pairs_v.at[pl.ds(c * _SG, _SG)],
                    sem,
                )
                for c in range(half_sg)
            ]
            for cp in copies:
                cp.wait()
            lax.fori_loop(
                0, (b_per_w // 2) // _L,
                select_block, h * (b_per_w // 2),
            )

        pltpu.sync_copy(rows_v, out_hbm.at[pl.ds(base, b_per_w)])

    return gather_kernel


def kernel(center_words, center_table):
    batch = center_words.shape[0]
    vocab, dim = center_table.shape
    idx = center_words.astype(jnp.int32)
    pairs = center_table.reshape(vocab // 2, 2 * dim)
    return _make_gather(batch, vocab, dim)(idx, pairs)


# speedup vs baseline: 1.6989x; 1.6989x over previous
import functools

import jax
import jax.numpy as jnp
from jax import lax
from jax.experimental import pallas as pl
from jax.experimental.pallas import tpu as pltpu
from jax.experimental.pallas import tpu_sc as plsc

_NC = 2
_NS = 16
_NW = _NC * _NS
_K = 16


def _make_sc_gather(batch, vocab, dim):
    b_per_w = batch // _NW
    n_step = b_per_w // _K
    mesh = plsc.VectorSubcoreMesh(core_axis_name="c", subcore_axis_name="s")

    @functools.partial(
        pl.kernel,
        mesh=mesh,
        out_type=jax.ShapeDtypeStruct((batch, dim), jnp.float32),
        scratch_types=[
            pltpu.VMEM((b_per_w,), jnp.int32),
            pltpu.VMEM((b_per_w, dim), jnp.float32),
            pltpu.SemaphoreType.DMA,
        ],
        compiler_params=pltpu.CompilerParams(skip_device_barrier=True),
    )
    def gather_kernel(idx_hbm, table_hbm, out_hbm, idx_v, rows_v, sem):
        wid = lax.axis_index("s") * _NC + lax.axis_index("c")
        base = wid * b_per_w
        pltpu.sync_copy(idx_hbm.at[pl.ds(base, b_per_w)], idx_v)

        def fire(s):
            idx_vec = idx_v[pl.ds(s * _K, _K)]
            for j in range(_K):
                row = idx_vec[j]
                pltpu.async_copy(
                    table_hbm.at[pl.ds(row, 1), :],
                    rows_v.at[pl.ds(s * _K + j, 1), :],
                    sem,
                )

        def drain():
            for _ in range(_K):
                pltpu.make_async_copy(
                    table_hbm.at[pl.ds(0, 1), :],
                    rows_v.at[pl.ds(0, 1), :],
                    sem,
                ).wait()

        fire(0)

        def step(s, carry):
            fire(s)
            drain()
            return carry

        lax.fori_loop(1, n_step, step, 0)
        drain()
        pltpu.sync_copy(rows_v, out_hbm.at[pl.ds(base, b_per_w)])

    return gather_kernel


def kernel(center_words, center_table):
    batch = center_words.shape[0]
    vocab, dim = center_table.shape
    idx = center_words.astype(jnp.int32)
    return _make_sc_gather(batch, vocab, dim)(idx, center_table)


# speedup vs baseline: 1.7145x; 1.0092x over previous
import functools

import jax
import jax.numpy as jnp
from jax import lax
from jax.experimental import pallas as pl
from jax.experimental.pallas import tpu as pltpu
from jax.experimental.pallas import tpu_sc as plsc

_NC = 2
_NS = 16
_NW = _NC * _NS
_K = 32


def _make_sc_gather(batch, vocab, dim):
    b_per_w = batch // _NW
    n_step = b_per_w // _K
    mesh = plsc.VectorSubcoreMesh(core_axis_name="c", subcore_axis_name="s")

    @functools.partial(
        pl.kernel,
        mesh=mesh,
        out_type=jax.ShapeDtypeStruct((batch, dim), jnp.float32),
        scratch_types=[
            pltpu.VMEM((b_per_w,), jnp.int32),
            pltpu.VMEM((b_per_w, dim), jnp.float32),
            pltpu.SemaphoreType.DMA,
        ],
        compiler_params=pltpu.CompilerParams(skip_device_barrier=True),
    )
    def gather_kernel(idx_hbm, table_hbm, out_hbm, idx_v, rows_v, sem):
        wid = lax.axis_index("s") * _NC + lax.axis_index("c")
        base = wid * b_per_w
        pltpu.sync_copy(idx_hbm.at[pl.ds(base, b_per_w)], idx_v)

        def fire(s):
            idx_vec = idx_v[pl.ds(s * _K, _K)]
            for j in range(_K):
                row = idx_vec[j]
                pltpu.async_copy(
                    table_hbm.at[pl.ds(row, 1), :],
                    rows_v.at[pl.ds(s * _K + j, 1), :],
                    sem,
                )

        def drain():
            for _ in range(_K):
                pltpu.make_async_copy(
                    table_hbm.at[pl.ds(0, 1), :],
                    rows_v.at[pl.ds(0, 1), :],
                    sem,
                ).wait()

        fire(0)

        def step(s, carry):
            fire(s)
            drain()
            return carry

        lax.fori_loop(1, n_step, step, 0)
        drain()
        pltpu.sync_copy(rows_v, out_hbm.at[pl.ds(base, b_per_w)])

    return gather_kernel


def kernel(center_words, center_table):
    batch = center_words.shape[0]
    vocab, dim = center_table.shape
    idx = center_words.astype(jnp.int32)
    return _make_sc_gather(batch, vocab, dim)(idx, center_table)
